# Initial kernel scaffold; baseline (speedup 1.0000x reference)
import jax
import jax.numpy as jnp
from jax.experimental import pallas as pl


def kernel(x, edge_index, W1, b1, W2, b2):
    raise NotImplementedError("write your pallas kernel here")


# speedup vs baseline: 5.8591x; 5.8591x over previous
import functools

import jax
import jax.numpy as jnp
from jax import lax
from jax.experimental import pallas as pl
from jax.experimental.pallas import tpu as pltpu
from jax.experimental.pallas import tpu_sc as plsc

N = 10000
NP = 10240
E = 320000
D = 128

NC = 2
NS = 16
NW = NC * NS
C = 128
NCH = 80
EPW = NCH * C
EP = NW * EPW
RPT = NP // NS

BM = 1000


_MESH = plsc.VectorSubcoreMesh(core_axis_name="c", subcore_axis_name="s")


def _deg_body(srci, dsti, ones_h, z128, out,
              sidx, didx, ones_v, deg_sh):
    c = lax.axis_index("c")
    s = lax.axis_index("s")
    wid = c * NS + s
    r0 = s * RPT
    pltpu.sync_copy(ones_h, ones_v)
    pltpu.sync_copy(z128, deg_sh.at[pl.ds(r0, RPT)])
    plsc.subcore_barrier()
    base = wid * EPW

    def step(i, carry):
        off = base + i * C
        pltpu.sync_copy(srci.at[pl.ds(off, C)], sidx.at[0])
        pltpu.sync_copy(dsti.at[pl.ds(off, C)], didx.at[0])
        pltpu.sync_copy(ones_v.at[0], deg_sh.at[sidx.at[0]], add=True)
        pltpu.sync_copy(ones_v.at[1], deg_sh.at[didx.at[0]], add=True)
        return carry

    lax.fori_loop(0, NCH, step, 0)
    plsc.subcore_barrier()
    pltpu.sync_copy(deg_sh.at[pl.ds(r0, RPT)], out.at[c, pl.ds(r0, RPT)])


_deg_kernel = functools.partial(
    pl.kernel,
    out_type=jax.ShapeDtypeStruct((NC, NP, D), jnp.float32),
    mesh=_MESH,
    scratch_types=[
        pltpu.VMEM((1, C), jnp.int32),
        pltpu.VMEM((1, C), jnp.int32),
        pltpu.VMEM((2, C, D), jnp.float32),
        pltpu.VMEM_SHARED((NP, D), jnp.float32),
    ],
)(_deg_body)


def _scat_body(h, srcg, dstg, z128, out,
               sidx, didx, rows, agg_sh, sem0, sem1):
    c = lax.axis_index("c")
    s = lax.axis_index("s")
    wid = c * NS + s
    r0 = s * RPT
    pltpu.sync_copy(z128, agg_sh.at[pl.ds(r0, RPT)])
    plsc.subcore_barrier()
    base = wid * EPW

    def step(j, carry):
        i0 = 2 * j
        off0 = base + i0 * C
        off1 = off0 + C
        pltpu.sync_copy(srcg.at[pl.ds(off0, C)], sidx.at[0])
        pltpu.sync_copy(dstg.at[pl.ds(off0, C)], didx.at[0])
        g0 = pltpu.async_copy(h.at[sidx.at[0]], rows.at[0], sem0)
        pltpu.sync_copy(srcg.at[pl.ds(off1, C)], sidx.at[1])
        pltpu.sync_copy(dstg.at[pl.ds(off1, C)], didx.at[1])
        g1 = pltpu.async_copy(h.at[sidx.at[1]], rows.at[1], sem1)
        g0.wait()
        pltpu.sync_copy(rows.at[0], agg_sh.at[didx.at[0]], add=True)
        g1.wait()
        pltpu.sync_copy(rows.at[1], agg_sh.at[didx.at[1]], add=True)
        return carry

    lax.fori_loop(0, NCH // 2, step, 0)
    plsc.subcore_barrier()
    pltpu.sync_copy(agg_sh.at[pl.ds(r0, RPT)], out.at[c, pl.ds(r0, RPT)])


_scat_kernel = functools.partial(
    pl.kernel,
    out_type=jax.ShapeDtypeStruct((NC, NP, D), jnp.float32),
    mesh=_MESH,
    scratch_types=[
        pltpu.VMEM((2, C), jnp.int32),
        pltpu.VMEM((2, C), jnp.int32),
        pltpu.VMEM((2, C, D), jnp.float32),
        pltpu.VMEM_SHARED((NP, D), jnp.float32),
        pltpu.SemaphoreType.DMA,
        pltpu.SemaphoreType.DMA,
    ],
)(_scat_body)


def _mm_body(x_ref, w_ref, o_ref):
    o_ref[...] = jnp.dot(x_ref[...], w_ref[...],
                         preferred_element_type=jnp.float32)


def _mm(x, w):
    return pl.pallas_call(
        _mm_body,
        grid=(N // BM,),
        in_specs=[pl.BlockSpec((BM, D), lambda i: (i, 0)),
                  pl.BlockSpec((D, D), lambda i: (0, 0))],
        out_specs=pl.BlockSpec((BM, D), lambda i: (i, 0)),
        out_shape=jax.ShapeDtypeStruct((N, D), jnp.float32),
    )(x, w)


def _scale_body(y_ref, deg_ref, h_ref, dsi_ref, ddi_ref):
    deg = deg_ref[0] + deg_ref[1]
    deg_s = deg[:, 0:1]
    deg_d = deg[:, 64:65]
    dsi = 1.0 / jnp.sqrt(jnp.maximum(deg_s, 1.0))
    ddi = 1.0 / jnp.sqrt(jnp.maximum(deg_d, 1.0))
    dsi_ref[...] = jnp.broadcast_to(dsi, dsi_ref.shape)
    ddi_ref[...] = jnp.broadcast_to(ddi, ddi_ref.shape)
    h_ref[...] = y_ref[...] * dsi


def _scale(y1, deg):
    return pl.pallas_call(
        _scale_body,
        grid=(N // BM,),
        in_specs=[pl.BlockSpec((BM, D), lambda i: (i, 0)),
                  pl.BlockSpec((NC, BM, D), lambda i: (0, i, 0))],
        out_specs=[pl.BlockSpec((BM, D), lambda i: (i, 0)),
                   pl.BlockSpec((BM, 16), lambda i: (i, 0)),
                   pl.BlockSpec((BM, 16), lambda i: (i, 0))],
        out_shape=[jax.ShapeDtypeStruct((N, D), jnp.float32),
                   jax.ShapeDtypeStruct((N, 16), jnp.float32),
                   jax.ShapeDtypeStruct((N, 16), jnp.float32)],
    )(y1, deg)


def _layer2_body(agg_ref, ddi_ref, b1_ref, w2_ref, dsi_ref, o_ref):
    a = agg_ref[0] + agg_ref[1]
    t = jnp.maximum(a * ddi_ref[..., :1] + b1_ref[...], 0.0)
    o_ref[...] = jnp.dot(t, w2_ref[...],
                         preferred_element_type=jnp.float32) * dsi_ref[..., :1]


def _layer2(agg1, ddi, b1_2d, w2, dsi):
    return pl.pallas_call(
        _layer2_body,
        grid=(N // BM,),
        in_specs=[pl.BlockSpec((NC, BM, D), lambda i: (0, i, 0)),
                  pl.BlockSpec((BM, 16), lambda i: (i, 0)),
                  pl.BlockSpec((1, D), lambda i: (0, 0)),
                  pl.BlockSpec((D, D), lambda i: (0, 0)),
                  pl.BlockSpec((BM, 16), lambda i: (i, 0))],
        out_specs=pl.BlockSpec((BM, D), lambda i: (i, 0)),
        out_shape=jax.ShapeDtypeStruct((N, D), jnp.float32),
    )(agg1, ddi, b1_2d, w2, dsi)


def _final_body(agg_ref, ddi_ref, b2_ref, o_ref):
    a = agg_ref[0] + agg_ref[1]
    o_ref[...] = a * ddi_ref[..., :1] + b2_ref[...]


def _final(agg2, ddi, b2_2d):
    return pl.pallas_call(
        _final_body,
        grid=(N // BM,),
        in_specs=[pl.BlockSpec((NC, BM, D), lambda i: (0, i, 0)),
                  pl.BlockSpec((BM, 16), lambda i: (i, 0)),
                  pl.BlockSpec((1, D), lambda i: (0, 0))],
        out_specs=pl.BlockSpec((BM, D), lambda i: (i, 0)),
        out_shape=jax.ShapeDtypeStruct((N, D), jnp.float32),
    )(agg2, ddi, b2_2d)


def kernel(x, edge_index, W1, b1, W2, b2):
    src = edge_index[0]
    dst = edge_index[1]
    pad = EP - E
    padi = jnp.arange(pad, dtype=jnp.int32)
    src_g = jnp.concatenate([src, padi % N])
    src_d = jnp.concatenate([src, N + (padi % 240)])
    dst_d = jnp.concatenate([dst, N + (padi % 240)])

    half = jnp.concatenate([jnp.ones((C, 64), jnp.float32),
                            jnp.zeros((C, 64), jnp.float32)], axis=1)
    ones2 = jnp.stack([half, 1.0 - half])
    z128 = jnp.zeros((RPT, D), jnp.float32)
    b1_2d = b1.reshape(1, D)
    b2_2d = b2.reshape(1, D)

    deg = _deg_kernel(src_d, dst_d, ones2, z128)
    y1 = _mm(x, W1)
    h1s, dsi, ddi = _scale(y1, deg)
    agg1 = _scat_kernel(h1s, src_g, dst_d, z128)
    h2s = _layer2(agg1, ddi, b1_2d, W2, dsi)
    agg2 = _scat_kernel(h2s, src_g, dst_d, z128)
    return _final(agg2, ddi, b2_2d)


# speedup vs baseline: 6.3497x; 1.0837x over previous
import functools

import jax
import jax.numpy as jnp
from jax import lax
from jax.experimental import pallas as pl
from jax.experimental.pallas import tpu as pltpu
from jax.experimental.pallas import tpu_sc as plsc

N = 10000
NP = 10112
E = 320000
D = 128

NC = 2
NS = 16
NW = NC * NS
C = 128
NCH = 81
EPW = NCH * C
EP = NW * EPW
RPT = NP // NS

BM = 1000


_MESH = plsc.VectorSubcoreMesh(core_axis_name="c", subcore_axis_name="s")


def _deg_body(srci, dsti, ones_h, z128, out,
              sidx, didx, ones_v, deg_sh):
    c = lax.axis_index("c")
    s = lax.axis_index("s")
    wid = c * NS + s
    r0 = s * RPT
    pltpu.sync_copy(ones_h, ones_v)
    pltpu.sync_copy(z128, deg_sh.at[pl.ds(r0, RPT)])
    plsc.subcore_barrier()
    base = wid * EPW

    def step(i, carry):
        off = base + i * C
        pltpu.sync_copy(srci.at[pl.ds(off, C)], sidx.at[0])
        pltpu.sync_copy(dsti.at[pl.ds(off, C)], didx.at[0])
        pltpu.sync_copy(ones_v.at[0], deg_sh.at[sidx.at[0]], add=True)
        pltpu.sync_copy(ones_v.at[1], deg_sh.at[didx.at[0]], add=True)
        return carry

    lax.fori_loop(0, NCH, step, 0)
    plsc.subcore_barrier()
    pltpu.sync_copy(deg_sh.at[pl.ds(r0, RPT)], out.at[c, pl.ds(r0, RPT)])


_deg_kernel = functools.partial(
    pl.kernel,
    out_type=jax.ShapeDtypeStruct((NC, NP, D), jnp.float32),
    mesh=_MESH,
    scratch_types=[
        pltpu.VMEM((1, C), jnp.int32),
        pltpu.VMEM((1, C), jnp.int32),
        pltpu.VMEM((2, C, D), jnp.float32),
        pltpu.VMEM_SHARED((NP, D), jnp.float32),
    ],
)(_deg_body)


_NBUF = 3


def _scat_body(h, srcg, dstg, z128, out,
               sidx, didx, rows, agg_sh, *sems):
    c = lax.axis_index("c")
    s = lax.axis_index("s")
    wid = c * NS + s
    r0 = s * RPT
    pltpu.sync_copy(z128, agg_sh.at[pl.ds(r0, RPT)])
    plsc.subcore_barrier()
    base = wid * EPW

    def fire(k, off):
        pltpu.sync_copy(srcg.at[pl.ds(off, C)], sidx.at[k])
        pltpu.sync_copy(dstg.at[pl.ds(off, C)], didx.at[k])
        return pltpu.async_copy(h.at[sidx.at[k]], rows.at[k], sems[k])

    def drain(k):
        pltpu.make_async_copy(h.at[sidx.at[k]], rows.at[k], sems[k]).wait()
        pltpu.sync_copy(rows.at[k], agg_sh.at[didx.at[k]], add=True)

    for k in range(_NBUF):
        fire(k, base + k * C)

    def step(j, carry):
        for k in range(_NBUF):
            i = _NBUF * j + k
            drain(k)
            fire(k, base + (i + _NBUF) * C)
        return carry

    lax.fori_loop(0, NCH // _NBUF - 1, step, 0)
    for k in range(_NBUF):
        drain(k)

    plsc.subcore_barrier()
    pltpu.sync_copy(agg_sh.at[pl.ds(r0, RPT)], out.at[c, pl.ds(r0, RPT)])


_scat_kernel = functools.partial(
    pl.kernel,
    out_type=jax.ShapeDtypeStruct((NC, NP, D), jnp.float32),
    mesh=_MESH,
    scratch_types=[
        pltpu.VMEM((_NBUF, C), jnp.int32),
        pltpu.VMEM((_NBUF, C), jnp.int32),
        pltpu.VMEM((_NBUF, C, D), jnp.float32),
        pltpu.VMEM_SHARED((NP, D), jnp.float32),
    ] + [pltpu.SemaphoreType.DMA] * _NBUF,
)(_scat_body)


def _mm_body(x_ref, w_ref, o_ref):
    o_ref[...] = jnp.dot(x_ref[...], w_ref[...],
                         preferred_element_type=jnp.float32)


def _mm(x, w):
    return pl.pallas_call(
        _mm_body,
        grid=(N // BM,),
        in_specs=[pl.BlockSpec((BM, D), lambda i: (i, 0)),
                  pl.BlockSpec((D, D), lambda i: (0, 0))],
        out_specs=pl.BlockSpec((BM, D), lambda i: (i, 0)),
        out_shape=jax.ShapeDtypeStruct((N, D), jnp.float32),
    )(x, w)


def _scale_body(y_ref, deg_ref, h_ref, dsi_ref, ddi_ref):
    deg = deg_ref[0] + deg_ref[1]
    deg_s = deg[:, 0:1]
    deg_d = deg[:, 64:65]
    dsi = 1.0 / jnp.sqrt(jnp.maximum(deg_s, 1.0))
    ddi = 1.0 / jnp.sqrt(jnp.maximum(deg_d, 1.0))
    dsi_ref[...] = jnp.broadcast_to(dsi, dsi_ref.shape)
    ddi_ref[...] = jnp.broadcast_to(ddi, ddi_ref.shape)
    h_ref[...] = y_ref[...] * dsi


def _scale(y1, deg):
    return pl.pallas_call(
        _scale_body,
        grid=(N // BM,),
        in_specs=[pl.BlockSpec((BM, D), lambda i: (i, 0)),
                  pl.BlockSpec((NC, BM, D), lambda i: (0, i, 0))],
        out_specs=[pl.BlockSpec((BM, D), lambda i: (i, 0)),
                   pl.BlockSpec((BM, 16), lambda i: (i, 0)),
                   pl.BlockSpec((BM, 16), lambda i: (i, 0))],
        out_shape=[jax.ShapeDtypeStruct((N, D), jnp.float32),
                   jax.ShapeDtypeStruct((N, 16), jnp.float32),
                   jax.ShapeDtypeStruct((N, 16), jnp.float32)],
    )(y1, deg)


def _layer2_body(agg_ref, ddi_ref, b1_ref, w2_ref, dsi_ref, o_ref):
    a = agg_ref[0] + agg_ref[1]
    t = jnp.maximum(a * ddi_ref[..., :1] + b1_ref[...], 0.0)
    o_ref[...] = jnp.dot(t, w2_ref[...],
                         preferred_element_type=jnp.float32) * dsi_ref[..., :1]


def _layer2(agg1, ddi, b1_2d, w2, dsi):
    return pl.pallas_call(
        _layer2_body,
        grid=(N // BM,),
        in_specs=[pl.BlockSpec((NC, BM, D), lambda i: (0, i, 0)),
                  pl.BlockSpec((BM, 16), lambda i: (i, 0)),
                  pl.BlockSpec((1, D), lambda i: (0, 0)),
                  pl.BlockSpec((D, D), lambda i: (0, 0)),
                  pl.BlockSpec((BM, 16), lambda i: (i, 0))],
        out_specs=pl.BlockSpec((BM, D), lambda i: (i, 0)),
        out_shape=jax.ShapeDtypeStruct((N, D), jnp.float32),
    )(agg1, ddi, b1_2d, w2, dsi)


def _final_body(agg_ref, ddi_ref, b2_ref, o_ref):
    a = agg_ref[0] + agg_ref[1]
    o_ref[...] = a * ddi_ref[..., :1] + b2_ref[...]


def _final(agg2, ddi, b2_2d):
    return pl.pallas_call(
        _final_body,
        grid=(N // BM,),
        in_specs=[pl.BlockSpec((NC, BM, D), lambda i: (0, i, 0)),
                  pl.BlockSpec((BM, 16), lambda i: (i, 0)),
                  pl.BlockSpec((1, D), lambda i: (0, 0))],
        out_specs=pl.BlockSpec((BM, D), lambda i: (i, 0)),
        out_shape=jax.ShapeDtypeStruct((N, D), jnp.float32),
    )(agg2, ddi, b2_2d)


def kernel(x, edge_index, W1, b1, W2, b2):
    src = edge_index[0]
    dst = edge_index[1]
    pad = EP - E
    padi = jnp.arange(pad, dtype=jnp.int32)
    src_g = jnp.concatenate([src, padi % N])
    src_d = jnp.concatenate([src, N + (padi % 112)])
    dst_d = jnp.concatenate([dst, N + (padi % 112)])

    half = jnp.concatenate([jnp.ones((C, 64), jnp.float32),
                            jnp.zeros((C, 64), jnp.float32)], axis=1)
    ones2 = jnp.stack([half, 1.0 - half])
    z128 = jnp.zeros((RPT, D), jnp.float32)
    b1_2d = b1.reshape(1, D)
    b2_2d = b2.reshape(1, D)

    deg = _deg_kernel(src_d, dst_d, ones2, z128)
    y1 = _mm(x, W1)
    h1s, dsi, ddi = _scale(y1, deg)
    agg1 = _scat_kernel(h1s, src_g, dst_d, z128)
    h2s = _layer2(agg1, ddi, b1_2d, W2, dsi)
    agg2 = _scat_kernel(h2s, src_g, dst_d, z128)
    return _final(agg2, ddi, b2_2d)


# speedup vs baseline: 8.4587x; 1.3321x over previous
import functools

import jax
import jax.numpy as jnp
from jax import lax
from jax.experimental import pallas as pl
from jax.experimental.pallas import tpu as pltpu
from jax.experimental.pallas import tpu_sc as plsc

N = 10000
NP = 10112
E = 320000
D = 128

NC = 2
NS = 16
NW = NC * NS
C = 128
NCH = 81
EPW = NCH * C
EP = NW * EPW
RPT = NP // NS

BM = 1000


_MESH = plsc.VectorSubcoreMesh(core_axis_name="c", subcore_axis_name="s")


NROW = 80


def _deg_body(srci, dsti, z128, iota_h, out_s, out_d,
              sv_buf, dv_buf, hist_s, hist_d, iota_v, deg_s_sh, deg_d_sh):
    c = lax.axis_index("c")
    s = lax.axis_index("s")
    wid = c * NS + s
    base = wid * EPW
    pltpu.sync_copy(srci.at[pl.ds(base, EPW)], sv_buf)
    pltpu.sync_copy(dsti.at[pl.ds(base, EPW)], dv_buf)
    pltpu.sync_copy(z128.at[pl.ds(0, NROW)], hist_s)
    pltpu.sync_copy(z128.at[pl.ds(0, NROW)], hist_d)
    pltpu.sync_copy(iota_h, iota_v)

    def step(k, carry):
        sv = sv_buf[pl.ds(k * 16, 16)]
        cnt_s, last_s = plsc.scan_count(sv)
        plsc.addupdate_scatter(
            hist_s,
            [lax.shift_right_logical(sv, 7), lax.bitwise_and(sv, 127)],
            cnt_s.astype(jnp.float32), mask=last_s)
        dv = dv_buf[pl.ds(k * 16, 16)]
        cnt_d, last_d = plsc.scan_count(dv)
        plsc.addupdate_scatter(
            hist_d,
            [lax.shift_right_logical(dv, 7), lax.bitwise_and(dv, 127)],
            cnt_d.astype(jnp.float32), mask=last_d)
        return carry

    lax.fori_loop(0, EPW // 16, step, 0)

    @pl.when(s == 0)
    def _():
        pltpu.sync_copy(z128.at[pl.ds(0, NROW)], deg_s_sh)
        pltpu.sync_copy(z128.at[pl.ds(0, NROW)], deg_d_sh)
    plsc.subcore_barrier()
    pltpu.sync_copy(hist_s, deg_s_sh.at[iota_v], add=True)
    pltpu.sync_copy(hist_d, deg_d_sh.at[iota_v], add=True)
    plsc.subcore_barrier()

    @pl.when(s == 0)
    def _():
        pltpu.sync_copy(deg_s_sh, out_s.at[c])
        pltpu.sync_copy(deg_d_sh, out_d.at[c])


_deg_kernel = functools.partial(
    pl.kernel,
    out_type=(jax.ShapeDtypeStruct((NC, NROW, 128), jnp.float32),
              jax.ShapeDtypeStruct((NC, NROW, 128), jnp.float32)),
    mesh=_MESH,
    compiler_params=pltpu.CompilerParams(needs_layout_passes=False),
    scratch_types=[
        pltpu.VMEM((EPW,), jnp.int32),
        pltpu.VMEM((EPW,), jnp.int32),
        pltpu.VMEM((NROW, 128), jnp.float32),
        pltpu.VMEM((NROW, 128), jnp.float32),
        pltpu.VMEM((NROW,), jnp.int32),
        pltpu.VMEM_SHARED((NROW, 128), jnp.float32),
        pltpu.VMEM_SHARED((NROW, 128), jnp.float32),
    ],
)(_deg_body)


_NBUF = 3


def _scat_body(h, srcg, dstg, z128, out,
               sidx, didx, rows, agg_sh, *sems):
    c = lax.axis_index("c")
    s = lax.axis_index("s")
    wid = c * NS + s
    r0 = s * RPT
    pltpu.sync_copy(z128, agg_sh.at[pl.ds(r0, RPT)])
    plsc.subcore_barrier()
    base = wid * EPW

    def fire(k, off):
        pltpu.sync_copy(srcg.at[pl.ds(off, C)], sidx.at[k])
        pltpu.sync_copy(dstg.at[pl.ds(off, C)], didx.at[k])
        return pltpu.async_copy(h.at[sidx.at[k]], rows.at[k], sems[k])

    def drain(k):
        pltpu.make_async_copy(h.at[sidx.at[k]], rows.at[k], sems[k]).wait()
        pltpu.sync_copy(rows.at[k], agg_sh.at[didx.at[k]], add=True)

    for k in range(_NBUF):
        fire(k, base + k * C)

    def step(j, carry):
        for k in range(_NBUF):
            i = _NBUF * j + k
            drain(k)
            fire(k, base + (i + _NBUF) * C)
        return carry

    lax.fori_loop(0, NCH // _NBUF - 1, step, 0)
    for k in range(_NBUF):
        drain(k)

    plsc.subcore_barrier()
    pltpu.sync_copy(agg_sh.at[pl.ds(r0, RPT)], out.at[c, pl.ds(r0, RPT)])


_scat_kernel = functools.partial(
    pl.kernel,
    out_type=jax.ShapeDtypeStruct((NC, NP, D), jnp.float32),
    mesh=_MESH,
    scratch_types=[
        pltpu.VMEM((_NBUF, C), jnp.int32),
        pltpu.VMEM((_NBUF, C), jnp.int32),
        pltpu.VMEM((_NBUF, C, D), jnp.float32),
        pltpu.VMEM_SHARED((NP, D), jnp.float32),
    ] + [pltpu.SemaphoreType.DMA] * _NBUF,
)(_scat_body)


def _mm_body(x_ref, w_ref, o_ref):
    o_ref[...] = jnp.dot(x_ref[...], w_ref[...],
                         preferred_element_type=jnp.float32)


def _mm(x, w):
    return pl.pallas_call(
        _mm_body,
        grid=(N // BM,),
        in_specs=[pl.BlockSpec((BM, D), lambda i: (i, 0)),
                  pl.BlockSpec((D, D), lambda i: (0, 0))],
        out_specs=pl.BlockSpec((BM, D), lambda i: (i, 0)),
        out_shape=jax.ShapeDtypeStruct((N, D), jnp.float32),
    )(x, w)


def _scale_body(y_ref, ds_ref, dd_ref, h_ref, dsi_ref, ddi_ref):
    deg_s = ds_ref[0] + ds_ref[1]
    deg_d = dd_ref[0] + dd_ref[1]
    dsi = 1.0 / jnp.sqrt(jnp.maximum(deg_s, 1.0))
    ddi = 1.0 / jnp.sqrt(jnp.maximum(deg_d, 1.0))
    dsi_ref[...] = jnp.broadcast_to(dsi, dsi_ref.shape)
    ddi_ref[...] = jnp.broadcast_to(ddi, ddi_ref.shape)
    h_ref[...] = y_ref[...] * dsi


def _scale(y1, deg_s_col, deg_d_col):
    return pl.pallas_call(
        _scale_body,
        grid=(N // BM,),
        in_specs=[pl.BlockSpec((BM, D), lambda i: (i, 0)),
                  pl.BlockSpec((NC, BM, 1), lambda i: (0, i, 0)),
                  pl.BlockSpec((NC, BM, 1), lambda i: (0, i, 0))],
        out_specs=[pl.BlockSpec((BM, D), lambda i: (i, 0)),
                   pl.BlockSpec((BM, 16), lambda i: (i, 0)),
                   pl.BlockSpec((BM, 16), lambda i: (i, 0))],
        out_shape=[jax.ShapeDtypeStruct((N, D), jnp.float32),
                   jax.ShapeDtypeStruct((N, 16), jnp.float32),
                   jax.ShapeDtypeStruct((N, 16), jnp.float32)],
    )(y1, deg_s_col, deg_d_col)


def _layer2_body(agg_ref, ddi_ref, b1_ref, w2_ref, dsi_ref, o_ref):
    a = agg_ref[0] + agg_ref[1]
    t = jnp.maximum(a * ddi_ref[..., :1] + b1_ref[...], 0.0)
    o_ref[...] = jnp.dot(t, w2_ref[...],
                         preferred_element_type=jnp.float32) * dsi_ref[..., :1]


def _layer2(agg1, ddi, b1_2d, w2, dsi):
    return pl.pallas_call(
        _layer2_body,
        grid=(N // BM,),
        in_specs=[pl.BlockSpec((NC, BM, D), lambda i: (0, i, 0)),
                  pl.BlockSpec((BM, 16), lambda i: (i, 0)),
                  pl.BlockSpec((1, D), lambda i: (0, 0)),
                  pl.BlockSpec((D, D), lambda i: (0, 0)),
                  pl.BlockSpec((BM, 16), lambda i: (i, 0))],
        out_specs=pl.BlockSpec((BM, D), lambda i: (i, 0)),
        out_shape=jax.ShapeDtypeStruct((N, D), jnp.float32),
    )(agg1, ddi, b1_2d, w2, dsi)


def _final_body(agg_ref, ddi_ref, b2_ref, o_ref):
    a = agg_ref[0] + agg_ref[1]
    o_ref[...] = a * ddi_ref[..., :1] + b2_ref[...]


def _final(agg2, ddi, b2_2d):
    return pl.pallas_call(
        _final_body,
        grid=(N // BM,),
        in_specs=[pl.BlockSpec((NC, BM, D), lambda i: (0, i, 0)),
                  pl.BlockSpec((BM, 16), lambda i: (i, 0)),
                  pl.BlockSpec((1, D), lambda i: (0, 0))],
        out_specs=pl.BlockSpec((BM, D), lambda i: (i, 0)),
        out_shape=jax.ShapeDtypeStruct((N, D), jnp.float32),
    )(agg2, ddi, b2_2d)


def kernel(x, edge_index, W1, b1, W2, b2):
    src = edge_index[0]
    dst = edge_index[1]
    pad = EP - E
    padi = jnp.arange(pad, dtype=jnp.int32)
    src_g = jnp.concatenate([src, padi % N])
    src_d = jnp.concatenate([src, N + (padi % 112)])
    dst_d = jnp.concatenate([dst, N + (padi % 112)])

    z128 = jnp.zeros((RPT, D), jnp.float32)
    iota_h = jnp.arange(NROW, dtype=jnp.int32)
    b1_2d = b1.reshape(1, D)
    b2_2d = b2.reshape(1, D)

    deg_s, deg_d = _deg_kernel(src_d, dst_d, z128, iota_h)
    y1 = _mm(x, W1)
    h1s, dsi, ddi = _scale(y1, deg_s.reshape(NC, NROW * 128, 1),
                           deg_d.reshape(NC, NROW * 128, 1))
    agg1 = _scat_kernel(h1s, src_g, dst_d, z128)
    h2s = _layer2(agg1, ddi, b1_2d, W2, dsi)
    agg2 = _scat_kernel(h2s, src_g, dst_d, z128)
    return _final(agg2, ddi, b2_2d)


# speedup vs baseline: 10.2253x; 1.2089x over previous
import functools

import jax
import jax.numpy as jnp
from jax import lax
from jax.experimental import pallas as pl
from jax.experimental.pallas import tpu as pltpu
from jax.experimental.pallas import tpu_sc as plsc

N = 10000
NP = 10112
E = 320000
D = 128

NC = 2
NS = 16
NW = NC * NS
C = 128
NCH = 80
EPW = NCH * C
EP = NW * EPW
RPT = NP // NS

BM = 1000


_MESH = plsc.VectorSubcoreMesh(core_axis_name="c", subcore_axis_name="s")


NROW = 80


def _deg_body(srci, dsti, z128, iota_h, out_s, out_d,
              sv_buf, dv_buf, hist_s, hist_d, iota_v, deg_s_sh, deg_d_sh):
    c = lax.axis_index("c")
    s = lax.axis_index("s")
    wid = c * NS + s
    base = wid * EPW
    pltpu.sync_copy(srci.at[pl.ds(base, EPW)], sv_buf)
    pltpu.sync_copy(dsti.at[pl.ds(base, EPW)], dv_buf)
    pltpu.sync_copy(z128.at[pl.ds(0, NROW)], hist_s)
    pltpu.sync_copy(z128.at[pl.ds(0, NROW)], hist_d)
    pltpu.sync_copy(iota_h, iota_v)

    def step(k, carry):
        sv = sv_buf[pl.ds(k * 16, 16)]
        cnt_s, last_s = plsc.scan_count(sv)
        plsc.addupdate_scatter(
            hist_s,
            [lax.shift_right_logical(sv, 7), lax.bitwise_and(sv, 127)],
            cnt_s.astype(jnp.float32), mask=last_s)
        dv = dv_buf[pl.ds(k * 16, 16)]
        cnt_d, last_d = plsc.scan_count(dv)
        plsc.addupdate_scatter(
            hist_d,
            [lax.shift_right_logical(dv, 7), lax.bitwise_and(dv, 127)],
            cnt_d.astype(jnp.float32), mask=last_d)
        return carry

    lax.fori_loop(0, EPW // 16, step, 0)

    @pl.when(s == 0)
    def _():
        pltpu.sync_copy(z128.at[pl.ds(0, NROW)], deg_s_sh)
        pltpu.sync_copy(z128.at[pl.ds(0, NROW)], deg_d_sh)
    plsc.subcore_barrier()
    pltpu.sync_copy(hist_s, deg_s_sh.at[iota_v], add=True)
    pltpu.sync_copy(hist_d, deg_d_sh.at[iota_v], add=True)
    plsc.subcore_barrier()

    @pl.when(s == 0)
    def _():
        pltpu.sync_copy(deg_s_sh, out_s.at[c])
        pltpu.sync_copy(deg_d_sh, out_d.at[c])


_deg_kernel = functools.partial(
    pl.kernel,
    out_type=(jax.ShapeDtypeStruct((NC, NROW, 128), jnp.float32),
              jax.ShapeDtypeStruct((NC, NROW, 128), jnp.float32)),
    mesh=_MESH,
    compiler_params=pltpu.CompilerParams(needs_layout_passes=False),
    scratch_types=[
        pltpu.VMEM((EPW,), jnp.int32),
        pltpu.VMEM((EPW,), jnp.int32),
        pltpu.VMEM((NROW, 128), jnp.float32),
        pltpu.VMEM((NROW, 128), jnp.float32),
        pltpu.VMEM((NROW,), jnp.int32),
        pltpu.VMEM_SHARED((NROW, 128), jnp.float32),
        pltpu.VMEM_SHARED((NROW, 128), jnp.float32),
    ],
)(_deg_body)


_NBUF = 2


NPH = 2
PCH = NCH // NPH


def _scat_body(h, srcg, dstg, z128, out,
               sidx, didx, rows, agg_sh, *sems):
    c = lax.axis_index("c")
    s = lax.axis_index("s")
    wid = c * NS + s
    r0 = s * RPT
    pltpu.sync_copy(z128, agg_sh.at[pl.ds(r0, RPT)])
    plsc.subcore_barrier()

    def fire(k, i):
        return pltpu.async_copy(h.at[sidx.at[i]], rows.at[k], sems[k])

    def drain(k, i):
        pltpu.make_async_copy(h.at[sidx.at[i]], rows.at[k], sems[k]).wait()
        pltpu.sync_copy(rows.at[k], agg_sh.at[didx.at[i]], add=True)

    for p in range(NPH):
        pltpu.sync_copy(srcg.at[wid, pl.ds(p * PCH, PCH)], sidx)
        pltpu.sync_copy(dstg.at[wid, pl.ds(p * PCH, PCH)], didx)
        for k in range(_NBUF):
            fire(k, k)

        def step(j, carry):
            for k in range(_NBUF):
                i = _NBUF * j + k
                drain(k, i)
                fire(k, i + _NBUF)
            return carry

        lax.fori_loop(0, PCH // _NBUF - 1, step, 0)
        for k in range(_NBUF):
            drain(k, PCH - _NBUF + k)

    plsc.subcore_barrier()
    pltpu.sync_copy(agg_sh.at[pl.ds(r0, RPT)], out.at[c, pl.ds(r0, RPT)])


_scat_kernel = functools.partial(
    pl.kernel,
    out_type=jax.ShapeDtypeStruct((NC, NP, D), jnp.float32),
    mesh=_MESH,
    scratch_types=[
        pltpu.VMEM((PCH, C), jnp.int32),
        pltpu.VMEM((PCH, C), jnp.int32),
        pltpu.VMEM((_NBUF, C, D), jnp.float32),
        pltpu.VMEM_SHARED((NP, D), jnp.float32),
    ] + [pltpu.SemaphoreType.DMA] * _NBUF,
)(_scat_body)


def _mm_body(x_ref, w_ref, o_ref):
    o_ref[...] = jnp.dot(x_ref[...], w_ref[...],
                         preferred_element_type=jnp.float32)


def _mm(x, w):
    return pl.pallas_call(
        _mm_body,
        grid=(N // BM,),
        in_specs=[pl.BlockSpec((BM, D), lambda i: (i, 0)),
                  pl.BlockSpec((D, D), lambda i: (0, 0))],
        out_specs=pl.BlockSpec((BM, D), lambda i: (i, 0)),
        out_shape=jax.ShapeDtypeStruct((N, D), jnp.float32),
    )(x, w)


def _scale_body(y_ref, ds_ref, dd_ref, h_ref, dsi_ref, ddi_ref):
    deg_s = ds_ref[0] + ds_ref[1]
    deg_d = dd_ref[0] + dd_ref[1]
    dsi = 1.0 / jnp.sqrt(jnp.maximum(deg_s, 1.0))
    ddi = 1.0 / jnp.sqrt(jnp.maximum(deg_d, 1.0))
    dsi_ref[...] = jnp.broadcast_to(dsi, dsi_ref.shape)
    ddi_ref[...] = jnp.broadcast_to(ddi, ddi_ref.shape)
    h_ref[...] = y_ref[...] * dsi


def _scale(y1, deg_s_col, deg_d_col):
    return pl.pallas_call(
        _scale_body,
        grid=(N // BM,),
        in_specs=[pl.BlockSpec((BM, D), lambda i: (i, 0)),
                  pl.BlockSpec((NC, BM, 1), lambda i: (0, i, 0)),
                  pl.BlockSpec((NC, BM, 1), lambda i: (0, i, 0))],
        out_specs=[pl.BlockSpec((BM, D), lambda i: (i, 0)),
                   pl.BlockSpec((BM, 16), lambda i: (i, 0)),
                   pl.BlockSpec((BM, 16), lambda i: (i, 0))],
        out_shape=[jax.ShapeDtypeStruct((N, D), jnp.float32),
                   jax.ShapeDtypeStruct((N, 16), jnp.float32),
                   jax.ShapeDtypeStruct((N, 16), jnp.float32)],
    )(y1, deg_s_col, deg_d_col)


def _layer2_body(agg_ref, ddi_ref, b1_ref, w2_ref, dsi_ref, o_ref):
    a = agg_ref[0] + agg_ref[1]
    t = jnp.maximum(a * ddi_ref[..., :1] + b1_ref[...], 0.0)
    o_ref[...] = jnp.dot(t, w2_ref[...],
                         preferred_element_type=jnp.float32) * dsi_ref[..., :1]


def _layer2(agg1, ddi, b1_2d, w2, dsi):
    return pl.pallas_call(
        _layer2_body,
        grid=(N // BM,),
        in_specs=[pl.BlockSpec((NC, BM, D), lambda i: (0, i, 0)),
                  pl.BlockSpec((BM, 16), lambda i: (i, 0)),
                  pl.BlockSpec((1, D), lambda i: (0, 0)),
                  pl.BlockSpec((D, D), lambda i: (0, 0)),
                  pl.BlockSpec((BM, 16), lambda i: (i, 0))],
        out_specs=pl.BlockSpec((BM, D), lambda i: (i, 0)),
        out_shape=jax.ShapeDtypeStruct((N, D), jnp.float32),
    )(agg1, ddi, b1_2d, w2, dsi)


def _final_body(agg_ref, ddi_ref, b2_ref, o_ref):
    a = agg_ref[0] + agg_ref[1]
    o_ref[...] = a * ddi_ref[..., :1] + b2_ref[...]


def _final(agg2, ddi, b2_2d):
    return pl.pallas_call(
        _final_body,
        grid=(N // BM,),
        in_specs=[pl.BlockSpec((NC, BM, D), lambda i: (0, i, 0)),
                  pl.BlockSpec((BM, 16), lambda i: (i, 0)),
                  pl.BlockSpec((1, D), lambda i: (0, 0))],
        out_specs=pl.BlockSpec((BM, D), lambda i: (i, 0)),
        out_shape=jax.ShapeDtypeStruct((N, D), jnp.float32),
    )(agg2, ddi, b2_2d)


def kernel(x, edge_index, W1, b1, W2, b2):
    src = edge_index[0]
    dst = edge_index[1]
    pad = EP - E
    padi = jnp.arange(pad, dtype=jnp.int32)
    src_g = jnp.concatenate([src, padi % N])
    src_d = jnp.concatenate([src, N + (padi % 112)])
    dst_d = jnp.concatenate([dst, N + (padi % 112)])

    z128 = jnp.zeros((RPT, D), jnp.float32)
    iota_h = jnp.arange(NROW, dtype=jnp.int32)
    b1_2d = b1.reshape(1, D)
    b2_2d = b2.reshape(1, D)

    src_g3 = src_g.reshape(NW, NCH, C)
    dst_d3 = dst_d.reshape(NW, NCH, C)

    deg_s, deg_d = _deg_kernel(src_d, dst_d, z128, iota_h)
    y1 = _mm(x, W1)
    h1s, dsi, ddi = _scale(y1, deg_s.reshape(NC, NROW * 128, 1),
                           deg_d.reshape(NC, NROW * 128, 1))
    agg1 = _scat_kernel(h1s, src_g3, dst_d3, z128)
    h2s = _layer2(agg1, ddi, b1_2d, W2, dsi)
    agg2 = _scat_kernel(h2s, src_g3, dst_d3, z128)
    return _final(agg2, ddi, b2_2d)


# speedup vs baseline: 10.4084x; 1.0179x over previous
import functools

import jax
import jax.numpy as jnp
from jax import lax
from jax.experimental import pallas as pl
from jax.experimental.pallas import tpu as pltpu
from jax.experimental.pallas import tpu_sc as plsc

N = 10000
NP = 10112
E = 320000
D = 128

NC = 2
NS = 16
NW = NC * NS
C = 128
NCH = 80
EPW = NCH * C
EP = NW * EPW
RPT = NP // NS

BM = 1000


_MESH = plsc.VectorSubcoreMesh(core_axis_name="c", subcore_axis_name="s")


NROW = 80


def _deg_body(srci, dsti, z128, iota_h, out_s, out_d,
              sv_buf, dv_buf, hist_s, hist_d, iota_v, deg_s_sh, deg_d_sh):
    c = lax.axis_index("c")
    s = lax.axis_index("s")
    wid = c * NS + s
    base = wid * EPW
    pltpu.sync_copy(srci.at[pl.ds(base, EPW)], sv_buf)
    pltpu.sync_copy(dsti.at[pl.ds(base, EPW)], dv_buf)
    pltpu.sync_copy(z128.at[pl.ds(0, NROW)], hist_s)
    pltpu.sync_copy(z128.at[pl.ds(0, NROW)], hist_d)
    pltpu.sync_copy(iota_h, iota_v)

    def step(k, carry):
        sv = sv_buf[pl.ds(k * 16, 16)]
        cnt_s, last_s = plsc.scan_count(sv)
        plsc.addupdate_scatter(
            hist_s,
            [lax.shift_right_logical(sv, 7), lax.bitwise_and(sv, 127)],
            cnt_s.astype(jnp.float32), mask=last_s)
        dv = dv_buf[pl.ds(k * 16, 16)]
        cnt_d, last_d = plsc.scan_count(dv)
        plsc.addupdate_scatter(
            hist_d,
            [lax.shift_right_logical(dv, 7), lax.bitwise_and(dv, 127)],
            cnt_d.astype(jnp.float32), mask=last_d)
        return carry

    lax.fori_loop(0, EPW // 16, step, 0)

    @pl.when(s == 0)
    def _():
        pltpu.sync_copy(z128.at[pl.ds(0, NROW)], deg_s_sh)
        pltpu.sync_copy(z128.at[pl.ds(0, NROW)], deg_d_sh)
    plsc.subcore_barrier()
    pltpu.sync_copy(hist_s, deg_s_sh.at[iota_v], add=True)
    pltpu.sync_copy(hist_d, deg_d_sh.at[iota_v], add=True)
    plsc.subcore_barrier()

    @pl.when(s == 0)
    def _():
        pltpu.sync_copy(deg_s_sh, out_s.at[c])
        pltpu.sync_copy(deg_d_sh, out_d.at[c])


_deg_kernel = functools.partial(
    pl.kernel,
    out_type=(jax.ShapeDtypeStruct((NC, NROW, 128), jnp.float32),
              jax.ShapeDtypeStruct((NC, NROW, 128), jnp.float32)),
    mesh=_MESH,
    compiler_params=pltpu.CompilerParams(needs_layout_passes=False),
    scratch_types=[
        pltpu.VMEM((EPW,), jnp.int32),
        pltpu.VMEM((EPW,), jnp.int32),
        pltpu.VMEM((NROW, 128), jnp.float32),
        pltpu.VMEM((NROW, 128), jnp.float32),
        pltpu.VMEM((NROW,), jnp.int32),
        pltpu.VMEM_SHARED((NROW, 128), jnp.float32),
        pltpu.VMEM_SHARED((NROW, 128), jnp.float32),
    ],
)(_deg_body)


_NBUF = 2


NPH = 2
PCH = NCH // NPH


def _scat_body(h, srcg, dstg, z128, out,
               sidx, didx, rows, agg_sh, *sems):
    c = lax.axis_index("c")
    s = lax.axis_index("s")
    wid = c * NS + s
    r0 = s * RPT
    pltpu.sync_copy(z128, agg_sh.at[pl.ds(r0, RPT)])
    plsc.subcore_barrier()

    def fire(k, i):
        return pltpu.async_copy(h.at[sidx.at[i]], rows.at[k], sems[k])

    def drain(k, i):
        pltpu.make_async_copy(h.at[sidx.at[i]], rows.at[k], sems[k]).wait()
        pltpu.sync_copy(rows.at[k], agg_sh.at[didx.at[i]], add=True)

    for p in range(NPH):
        pltpu.sync_copy(srcg.at[wid, pl.ds(p * PCH, PCH)], sidx)
        pltpu.sync_copy(dstg.at[wid, pl.ds(p * PCH, PCH)], didx)
        for k in range(_NBUF):
            fire(k, k)

        def step(j, carry):
            for k in range(_NBUF):
                i = _NBUF * j + k
                drain(k, i)
                fire(k, i + _NBUF)
            return carry

        lax.fori_loop(0, PCH // _NBUF - 1, step, 0)
        for k in range(_NBUF):
            drain(k, PCH - _NBUF + k)

    plsc.subcore_barrier()
    pltpu.sync_copy(agg_sh.at[pl.ds(r0, RPT)], out.at[c, pl.ds(r0, RPT)])


_scat_kernel = functools.partial(
    pl.kernel,
    out_type=jax.ShapeDtypeStruct((NC, NP, D), jnp.float32),
    mesh=_MESH,
    scratch_types=[
        pltpu.VMEM((PCH, C), jnp.int32),
        pltpu.VMEM((PCH, C), jnp.int32),
        pltpu.VMEM((_NBUF, C, D), jnp.float32),
        pltpu.VMEM_SHARED((NP, D), jnp.float32),
    ] + [pltpu.SemaphoreType.DMA] * _NBUF,
)(_scat_body)


def _scale_body(x_ref, w_ref, ds_ref, dd_ref, h_ref, dsi_ref, ddi_ref):
    deg_s = ds_ref[0] + ds_ref[1]
    deg_d = dd_ref[0] + dd_ref[1]
    dsi = 1.0 / jnp.sqrt(jnp.maximum(deg_s, 1.0))
    ddi = 1.0 / jnp.sqrt(jnp.maximum(deg_d, 1.0))
    dsi_ref[...] = jnp.broadcast_to(dsi, dsi_ref.shape)
    ddi_ref[...] = jnp.broadcast_to(ddi, ddi_ref.shape)
    y = jnp.dot(x_ref[...], w_ref[...], preferred_element_type=jnp.float32)
    h_ref[...] = y * dsi


def _scale(x, w1, deg_s_col, deg_d_col):
    return pl.pallas_call(
        _scale_body,
        grid=(N // BM,),
        in_specs=[pl.BlockSpec((BM, D), lambda i: (i, 0)),
                  pl.BlockSpec((D, D), lambda i: (0, 0)),
                  pl.BlockSpec((NC, BM, 1), lambda i: (0, i, 0)),
                  pl.BlockSpec((NC, BM, 1), lambda i: (0, i, 0))],
        out_specs=[pl.BlockSpec((BM, D), lambda i: (i, 0)),
                   pl.BlockSpec((BM, 16), lambda i: (i, 0)),
                   pl.BlockSpec((BM, 16), lambda i: (i, 0))],
        out_shape=[jax.ShapeDtypeStruct((N, D), jnp.float32),
                   jax.ShapeDtypeStruct((N, 16), jnp.float32),
                   jax.ShapeDtypeStruct((N, 16), jnp.float32)],
    )(x, w1, deg_s_col, deg_d_col)


def _layer2_body(agg_ref, ddi_ref, b1_ref, w2_ref, dsi_ref, o_ref):
    a = agg_ref[0] + agg_ref[1]
    t = jnp.maximum(a * ddi_ref[..., :1] + b1_ref[...], 0.0)
    o_ref[...] = jnp.dot(t, w2_ref[...],
                         preferred_element_type=jnp.float32) * dsi_ref[..., :1]


def _layer2(agg1, ddi, b1_2d, w2, dsi):
    return pl.pallas_call(
        _layer2_body,
        grid=(N // BM,),
        in_specs=[pl.BlockSpec((NC, BM, D), lambda i: (0, i, 0)),
                  pl.BlockSpec((BM, 16), lambda i: (i, 0)),
                  pl.BlockSpec((1, D), lambda i: (0, 0)),
                  pl.BlockSpec((D, D), lambda i: (0, 0)),
                  pl.BlockSpec((BM, 16), lambda i: (i, 0))],
        out_specs=pl.BlockSpec((BM, D), lambda i: (i, 0)),
        out_shape=jax.ShapeDtypeStruct((N, D), jnp.float32),
    )(agg1, ddi, b1_2d, w2, dsi)


def _final_body(agg_ref, ddi_ref, b2_ref, o_ref):
    a = agg_ref[0] + agg_ref[1]
    o_ref[...] = a * ddi_ref[..., :1] + b2_ref[...]


def _final(agg2, ddi, b2_2d):
    return pl.pallas_call(
        _final_body,
        grid=(N // BM,),
        in_specs=[pl.BlockSpec((NC, BM, D), lambda i: (0, i, 0)),
                  pl.BlockSpec((BM, 16), lambda i: (i, 0)),
                  pl.BlockSpec((1, D), lambda i: (0, 0))],
        out_specs=pl.BlockSpec((BM, D), lambda i: (i, 0)),
        out_shape=jax.ShapeDtypeStruct((N, D), jnp.float32),
    )(agg2, ddi, b2_2d)


def kernel(x, edge_index, W1, b1, W2, b2):
    src = edge_index[0]
    dst = edge_index[1]
    pad = EP - E
    padi = jnp.arange(pad, dtype=jnp.int32)
    src_g = jnp.concatenate([src, padi % N])
    src_d = jnp.concatenate([src, N + (padi % 112)])
    dst_d = jnp.concatenate([dst, N + (padi % 112)])

    z128 = jnp.zeros((RPT, D), jnp.float32)
    iota_h = jnp.arange(NROW, dtype=jnp.int32)
    b1_2d = b1.reshape(1, D)
    b2_2d = b2.reshape(1, D)

    src_g3 = src_g.reshape(NW, NCH, C)
    dst_d3 = dst_d.reshape(NW, NCH, C)

    deg_s, deg_d = _deg_kernel(src_d, dst_d, z128, iota_h)
    h1s, dsi, ddi = _scale(x, W1, deg_s.reshape(NC, NROW * 128, 1),
                           deg_d.reshape(NC, NROW * 128, 1))
    agg1 = _scat_kernel(h1s, src_g3, dst_d3, z128)
    h2s = _layer2(agg1, ddi, b1_2d, W2, dsi)
    agg2 = _scat_kernel(h2s, src_g3, dst_d3, z128)
    return _final(agg2, ddi, b2_2d)


# speedup vs baseline: 10.7528x; 1.0331x over previous
import functools

import jax
import jax.numpy as jnp
from jax import lax
from jax.experimental import pallas as pl
from jax.experimental.pallas import tpu as pltpu
from jax.experimental.pallas import tpu_sc as plsc

N = 10000
NP = 10112
E = 320000
D = 128

NC = 2
NS = 16
NW = NC * NS
C = 64
NCH = 160
EPW = NCH * C
EP = NW * EPW
RPT = NP // NS

BM = 1000


_MESH = plsc.VectorSubcoreMesh(core_axis_name="c", subcore_axis_name="s")


NROW = 80


def _deg_body(srci, dsti, z128, iota_h, out_s, out_d,
              sv_buf, dv_buf, hist_s, hist_d, iota_v, deg_s_sh, deg_d_sh):
    c = lax.axis_index("c")
    s = lax.axis_index("s")
    wid = c * NS + s
    base = wid * EPW
    pltpu.sync_copy(srci.at[pl.ds(base, EPW)], sv_buf)
    pltpu.sync_copy(dsti.at[pl.ds(base, EPW)], dv_buf)
    pltpu.sync_copy(z128.at[pl.ds(0, NROW)], hist_s)
    pltpu.sync_copy(z128.at[pl.ds(0, NROW)], hist_d)
    pltpu.sync_copy(iota_h, iota_v)

    def step(k, carry):
        sv = sv_buf[pl.ds(k * 16, 16)]
        cnt_s, last_s = plsc.scan_count(sv)
        plsc.addupdate_scatter(
            hist_s,
            [lax.shift_right_logical(sv, 7), lax.bitwise_and(sv, 127)],
            cnt_s.astype(jnp.float32), mask=last_s)
        dv = dv_buf[pl.ds(k * 16, 16)]
        cnt_d, last_d = plsc.scan_count(dv)
        plsc.addupdate_scatter(
            hist_d,
            [lax.shift_right_logical(dv, 7), lax.bitwise_and(dv, 127)],
            cnt_d.astype(jnp.float32), mask=last_d)
        return carry

    lax.fori_loop(0, EPW // 16, step, 0)

    @pl.when(s == 0)
    def _():
        pltpu.sync_copy(z128.at[pl.ds(0, NROW)], deg_s_sh)
        pltpu.sync_copy(z128.at[pl.ds(0, NROW)], deg_d_sh)
    plsc.subcore_barrier()
    pltpu.sync_copy(hist_s, deg_s_sh.at[iota_v], add=True)
    pltpu.sync_copy(hist_d, deg_d_sh.at[iota_v], add=True)
    plsc.subcore_barrier()

    @pl.when(s == 0)
    def _():
        pltpu.sync_copy(deg_s_sh, out_s.at[c])
        pltpu.sync_copy(deg_d_sh, out_d.at[c])


_deg_kernel = functools.partial(
    pl.kernel,
    out_type=(jax.ShapeDtypeStruct((NC, NROW, 128), jnp.float32),
              jax.ShapeDtypeStruct((NC, NROW, 128), jnp.float32)),
    mesh=_MESH,
    compiler_params=pltpu.CompilerParams(needs_layout_passes=False),
    scratch_types=[
        pltpu.VMEM((EPW,), jnp.int32),
        pltpu.VMEM((EPW,), jnp.int32),
        pltpu.VMEM((NROW, 128), jnp.float32),
        pltpu.VMEM((NROW, 128), jnp.float32),
        pltpu.VMEM((NROW,), jnp.int32),
        pltpu.VMEM_SHARED((NROW, 128), jnp.float32),
        pltpu.VMEM_SHARED((NROW, 128), jnp.float32),
    ],
)(_deg_body)


_NBUF = 4


NPH = 4
PCH = NCH // NPH


def _scat_body(h, srcg, dstg, z128, out,
               sidx, didx, rows, agg_sh, *sems):
    c = lax.axis_index("c")
    s = lax.axis_index("s")
    wid = c * NS + s
    r0 = s * RPT
    pltpu.sync_copy(z128, agg_sh.at[pl.ds(r0, RPT)])
    plsc.subcore_barrier()

    def fire(k, i):
        return pltpu.async_copy(h.at[sidx.at[i]], rows.at[k], sems[k])

    def drain(k, i):
        pltpu.make_async_copy(h.at[sidx.at[i]], rows.at[k], sems[k]).wait()
        pltpu.sync_copy(rows.at[k], agg_sh.at[didx.at[i]], add=True)

    for p in range(NPH):
        pltpu.sync_copy(srcg.at[wid, pl.ds(p * PCH, PCH)], sidx)
        pltpu.sync_copy(dstg.at[wid, pl.ds(p * PCH, PCH)], didx)
        for k in range(_NBUF):
            fire(k, k)

        def step(j, carry):
            for k in range(_NBUF):
                i = _NBUF * j + k
                drain(k, i)
                fire(k, i + _NBUF)
            return carry

        lax.fori_loop(0, PCH // _NBUF - 1, step, 0)
        for k in range(_NBUF):
            drain(k, PCH - _NBUF + k)

    plsc.subcore_barrier()
    pltpu.sync_copy(agg_sh.at[pl.ds(r0, RPT)], out.at[c, pl.ds(r0, RPT)])


_scat_kernel = functools.partial(
    pl.kernel,
    out_type=jax.ShapeDtypeStruct((NC, NP, D), jnp.float32),
    mesh=_MESH,
    scratch_types=[
        pltpu.VMEM((PCH, C), jnp.int32),
        pltpu.VMEM((PCH, C), jnp.int32),
        pltpu.VMEM((_NBUF, C, D), jnp.float32),
        pltpu.VMEM_SHARED((NP, D), jnp.float32),
    ] + [pltpu.SemaphoreType.DMA] * _NBUF,
)(_scat_body)


def _scale_body(x_ref, w_ref, ds_ref, dd_ref, h_ref, dsi_ref, ddi_ref):
    deg_s = ds_ref[0] + ds_ref[1]
    deg_d = dd_ref[0] + dd_ref[1]
    dsi = 1.0 / jnp.sqrt(jnp.maximum(deg_s, 1.0))
    ddi = 1.0 / jnp.sqrt(jnp.maximum(deg_d, 1.0))
    dsi_ref[...] = jnp.broadcast_to(dsi, dsi_ref.shape)
    ddi_ref[...] = jnp.broadcast_to(ddi, ddi_ref.shape)
    y = jnp.dot(x_ref[...], w_ref[...], preferred_element_type=jnp.float32)
    h_ref[...] = y * dsi


def _scale(x, w1, deg_s_col, deg_d_col):
    return pl.pallas_call(
        _scale_body,
        grid=(N // BM,),
        in_specs=[pl.BlockSpec((BM, D), lambda i: (i, 0)),
                  pl.BlockSpec((D, D), lambda i: (0, 0)),
                  pl.BlockSpec((NC, BM, 1), lambda i: (0, i, 0)),
                  pl.BlockSpec((NC, BM, 1), lambda i: (0, i, 0))],
        out_specs=[pl.BlockSpec((BM, D), lambda i: (i, 0)),
                   pl.BlockSpec((BM, 16), lambda i: (i, 0)),
                   pl.BlockSpec((BM, 16), lambda i: (i, 0))],
        out_shape=[jax.ShapeDtypeStruct((N, D), jnp.float32),
                   jax.ShapeDtypeStruct((N, 16), jnp.float32),
                   jax.ShapeDtypeStruct((N, 16), jnp.float32)],
    )(x, w1, deg_s_col, deg_d_col)


def _layer2_body(agg_ref, ddi_ref, b1_ref, w2_ref, dsi_ref, o_ref):
    a = agg_ref[0] + agg_ref[1]
    t = jnp.maximum(a * ddi_ref[..., :1] + b1_ref[...], 0.0)
    o_ref[...] = jnp.dot(t, w2_ref[...],
                         preferred_element_type=jnp.float32) * dsi_ref[..., :1]


def _layer2(agg1, ddi, b1_2d, w2, dsi):
    return pl.pallas_call(
        _layer2_body,
        grid=(N // BM,),
        in_specs=[pl.BlockSpec((NC, BM, D), lambda i: (0, i, 0)),
                  pl.BlockSpec((BM, 16), lambda i: (i, 0)),
                  pl.BlockSpec((1, D), lambda i: (0, 0)),
                  pl.BlockSpec((D, D), lambda i: (0, 0)),
                  pl.BlockSpec((BM, 16), lambda i: (i, 0))],
        out_specs=pl.BlockSpec((BM, D), lambda i: (i, 0)),
        out_shape=jax.ShapeDtypeStruct((N, D), jnp.float32),
    )(agg1, ddi, b1_2d, w2, dsi)


def _final_body(agg_ref, ddi_ref, b2_ref, o_ref):
    a = agg_ref[0] + agg_ref[1]
    o_ref[...] = a * ddi_ref[..., :1] + b2_ref[...]


def _final(agg2, ddi, b2_2d):
    return pl.pallas_call(
        _final_body,
        grid=(N // BM,),
        in_specs=[pl.BlockSpec((NC, BM, D), lambda i: (0, i, 0)),
                  pl.BlockSpec((BM, 16), lambda i: (i, 0)),
                  pl.BlockSpec((1, D), lambda i: (0, 0))],
        out_specs=pl.BlockSpec((BM, D), lambda i: (i, 0)),
        out_shape=jax.ShapeDtypeStruct((N, D), jnp.float32),
    )(agg2, ddi, b2_2d)


def kernel(x, edge_index, W1, b1, W2, b2):
    src = edge_index[0]
    dst = edge_index[1]
    pad = EP - E
    padi = jnp.arange(pad, dtype=jnp.int32)
    src_g = jnp.concatenate([src, padi % N])
    src_d = jnp.concatenate([src, N + (padi % 112)])
    dst_d = jnp.concatenate([dst, N + (padi % 112)])

    z128 = jnp.zeros((RPT, D), jnp.float32)
    iota_h = jnp.arange(NROW, dtype=jnp.int32)
    b1_2d = b1.reshape(1, D)
    b2_2d = b2.reshape(1, D)

    src_g3 = src_g.reshape(NW, NCH, C)
    dst_d3 = dst_d.reshape(NW, NCH, C)

    deg_s, deg_d = _deg_kernel(src_d, dst_d, z128, iota_h)
    h1s, dsi, ddi = _scale(x, W1, deg_s.reshape(NC, NROW * 128, 1),
                           deg_d.reshape(NC, NROW * 128, 1))
    agg1 = _scat_kernel(h1s, src_g3, dst_d3, z128)
    h2s = _layer2(agg1, ddi, b1_2d, W2, dsi)
    agg2 = _scat_kernel(h2s, src_g3, dst_d3, z128)
    return _final(agg2, ddi, b2_2d)


# speedup vs baseline: 10.9403x; 1.0174x over previous
import functools

import jax
import jax.numpy as jnp
from jax import lax
from jax.experimental import pallas as pl
from jax.experimental.pallas import tpu as pltpu
from jax.experimental.pallas import tpu_sc as plsc

N = 10000
NP = 10112
E = 320000
D = 128

NC = 2
NS = 16
NW = NC * NS
C = 64
NCH = 160
EPW = NCH * C
EP = NW * EPW
RPT = NP // NS

BM = 1000


_MESH = plsc.VectorSubcoreMesh(core_axis_name="c", subcore_axis_name="s")


NROW = 80


def _deg_body(srci, dsti, z128, iota_h, out_s, out_d,
              sv_buf, dv_buf, hist_s, hist_d, iota_v, deg_s_sh, deg_d_sh):
    c = lax.axis_index("c")
    s = lax.axis_index("s")
    wid = c * NS + s
    base = wid * EPW
    pltpu.sync_copy(srci.at[pl.ds(base, EPW)], sv_buf)
    pltpu.sync_copy(dsti.at[pl.ds(base, EPW)], dv_buf)
    pltpu.sync_copy(z128.at[pl.ds(0, NROW)], hist_s)
    pltpu.sync_copy(z128.at[pl.ds(0, NROW)], hist_d)
    pltpu.sync_copy(iota_h, iota_v)

    def step(k, carry):
        sv = sv_buf[pl.ds(k * 16, 16)]
        cnt_s, last_s = plsc.scan_count(sv)
        plsc.addupdate_scatter(
            hist_s,
            [lax.shift_right_logical(sv, 7), lax.bitwise_and(sv, 127)],
            cnt_s.astype(jnp.float32), mask=last_s)
        dv = dv_buf[pl.ds(k * 16, 16)]
        cnt_d, last_d = plsc.scan_count(dv)
        plsc.addupdate_scatter(
            hist_d,
            [lax.shift_right_logical(dv, 7), lax.bitwise_and(dv, 127)],
            cnt_d.astype(jnp.float32), mask=last_d)
        return carry

    lax.fori_loop(0, EPW // 16, step, 0)

    @pl.when(s == 0)
    def _():
        pltpu.sync_copy(z128.at[pl.ds(0, NROW)], deg_s_sh)
        pltpu.sync_copy(z128.at[pl.ds(0, NROW)], deg_d_sh)
    plsc.subcore_barrier()
    pltpu.sync_copy(hist_s, deg_s_sh.at[iota_v], add=True)
    pltpu.sync_copy(hist_d, deg_d_sh.at[iota_v], add=True)
    plsc.subcore_barrier()

    @pl.when(s == 0)
    def _():
        pltpu.sync_copy(deg_s_sh, out_s.at[c])
        pltpu.sync_copy(deg_d_sh, out_d.at[c])


_deg_kernel = functools.partial(
    pl.kernel,
    out_type=(jax.ShapeDtypeStruct((NC, NROW, 128), jnp.float32),
              jax.ShapeDtypeStruct((NC, NROW, 128), jnp.float32)),
    mesh=_MESH,
    compiler_params=pltpu.CompilerParams(needs_layout_passes=False),
    scratch_types=[
        pltpu.VMEM((EPW,), jnp.int32),
        pltpu.VMEM((EPW,), jnp.int32),
        pltpu.VMEM((NROW, 128), jnp.float32),
        pltpu.VMEM((NROW, 128), jnp.float32),
        pltpu.VMEM((NROW,), jnp.int32),
        pltpu.VMEM_SHARED((NROW, 128), jnp.float32),
        pltpu.VMEM_SHARED((NROW, 128), jnp.float32),
    ],
)(_deg_body)


_NBUF = 4


NPH = 5
PCH = NCH // NPH


def _scat_body(h, srcg, dstg, z128, out,
               sidx, didx, rows, agg_sh, *sems):
    gsems = sems[:_NBUF]
    ssems = sems[_NBUF:]
    c = lax.axis_index("c")
    s = lax.axis_index("s")
    wid = c * NS + s
    r0 = s * RPT
    pltpu.sync_copy(z128, agg_sh.at[pl.ds(r0, RPT)])
    plsc.subcore_barrier()

    def stage(q, p):
        return (pltpu.async_copy(srcg.at[wid, pl.ds(p * PCH, PCH)],
                                 sidx.at[q], ssems[q]),
                pltpu.async_copy(dstg.at[wid, pl.ds(p * PCH, PCH)],
                                 didx.at[q], ssems[q]))

    def stage_wait(q, p):
        pltpu.make_async_copy(srcg.at[wid, pl.ds(p * PCH, PCH)],
                              sidx.at[q], ssems[q]).wait()
        pltpu.make_async_copy(dstg.at[wid, pl.ds(p * PCH, PCH)],
                              didx.at[q], ssems[q]).wait()

    def fire(k, q, i):
        return pltpu.async_copy(h.at[sidx.at[q, i]], rows.at[k], gsems[k])

    def drain(k, q, i):
        pltpu.make_async_copy(h.at[sidx.at[q, i]], rows.at[k],
                              gsems[k]).wait()
        pltpu.sync_copy(rows.at[k], agg_sh.at[didx.at[q, i]], add=True)

    stage(0, 0)
    stage_wait(0, 0)
    for p in range(NPH):
        q = p % 2
        if p + 1 < NPH:
            stage(1 - q, p + 1)
        for k in range(_NBUF):
            fire(k, q, k)

        def step(j, carry, q=q):
            for k in range(_NBUF):
                i = _NBUF * j + k
                drain(k, q, i)
                fire(k, q, i + _NBUF)
            return carry

        lax.fori_loop(0, PCH // _NBUF - 1, step, 0)
        for k in range(_NBUF):
            drain(k, q, PCH - _NBUF + k)
        if p + 1 < NPH:
            stage_wait(1 - q, p + 1)

    plsc.subcore_barrier()
    pltpu.sync_copy(agg_sh.at[pl.ds(r0, RPT)], out.at[c, pl.ds(r0, RPT)])


_scat_kernel = functools.partial(
    pl.kernel,
    out_type=jax.ShapeDtypeStruct((NC, NP, D), jnp.float32),
    mesh=_MESH,
    scratch_types=[
        pltpu.VMEM((2, PCH, C), jnp.int32),
        pltpu.VMEM((2, PCH, C), jnp.int32),
        pltpu.VMEM((_NBUF, C, D), jnp.float32),
        pltpu.VMEM_SHARED((NP, D), jnp.float32),
    ] + [pltpu.SemaphoreType.DMA] * (_NBUF + 2),
)(_scat_body)


def _scale_body(x_ref, w_ref, ds_ref, dd_ref, h_ref, dsi_ref, ddi_ref):
    deg_s = ds_ref[0] + ds_ref[1]
    deg_d = dd_ref[0] + dd_ref[1]
    dsi = 1.0 / jnp.sqrt(jnp.maximum(deg_s, 1.0))
    ddi = 1.0 / jnp.sqrt(jnp.maximum(deg_d, 1.0))
    dsi_ref[...] = jnp.broadcast_to(dsi, dsi_ref.shape)
    ddi_ref[...] = jnp.broadcast_to(ddi, ddi_ref.shape)
    y = jnp.dot(x_ref[...], w_ref[...], preferred_element_type=jnp.float32)
    h_ref[...] = y * dsi


def _scale(x, w1, deg_s_col, deg_d_col):
    return pl.pallas_call(
        _scale_body,
        grid=(N // BM,),
        in_specs=[pl.BlockSpec((BM, D), lambda i: (i, 0)),
                  pl.BlockSpec((D, D), lambda i: (0, 0)),
                  pl.BlockSpec((NC, BM, 1), lambda i: (0, i, 0)),
                  pl.BlockSpec((NC, BM, 1), lambda i: (0, i, 0))],
        out_specs=[pl.BlockSpec((BM, D), lambda i: (i, 0)),
                   pl.BlockSpec((BM, 16), lambda i: (i, 0)),
                   pl.BlockSpec((BM, 16), lambda i: (i, 0))],
        out_shape=[jax.ShapeDtypeStruct((N, D), jnp.float32),
                   jax.ShapeDtypeStruct((N, 16), jnp.float32),
                   jax.ShapeDtypeStruct((N, 16), jnp.float32)],
    )(x, w1, deg_s_col, deg_d_col)


def _layer2_body(agg_ref, ddi_ref, b1_ref, w2_ref, dsi_ref, o_ref):
    a = agg_ref[0] + agg_ref[1]
    t = jnp.maximum(a * ddi_ref[..., :1] + b1_ref[...], 0.0)
    o_ref[...] = jnp.dot(t, w2_ref[...],
                         preferred_element_type=jnp.float32) * dsi_ref[..., :1]


def _layer2(agg1, ddi, b1_2d, w2, dsi):
    return pl.pallas_call(
        _layer2_body,
        grid=(N // BM,),
        in_specs=[pl.BlockSpec((NC, BM, D), lambda i: (0, i, 0)),
                  pl.BlockSpec((BM, 16), lambda i: (i, 0)),
                  pl.BlockSpec((1, D), lambda i: (0, 0)),
                  pl.BlockSpec((D, D), lambda i: (0, 0)),
                  pl.BlockSpec((BM, 16), lambda i: (i, 0))],
        out_specs=pl.BlockSpec((BM, D), lambda i: (i, 0)),
        out_shape=jax.ShapeDtypeStruct((N, D), jnp.float32),
    )(agg1, ddi, b1_2d, w2, dsi)


def _final_body(agg_ref, ddi_ref, b2_ref, o_ref):
    a = agg_ref[0] + agg_ref[1]
    o_ref[...] = a * ddi_ref[..., :1] + b2_ref[...]


def _final(agg2, ddi, b2_2d):
    return pl.pallas_call(
        _final_body,
        grid=(N // BM,),
        in_specs=[pl.BlockSpec((NC, BM, D), lambda i: (0, i, 0)),
                  pl.BlockSpec((BM, 16), lambda i: (i, 0)),
                  pl.BlockSpec((1, D), lambda i: (0, 0))],
        out_specs=pl.BlockSpec((BM, D), lambda i: (i, 0)),
        out_shape=jax.ShapeDtypeStruct((N, D), jnp.float32),
    )(agg2, ddi, b2_2d)


def kernel(x, edge_index, W1, b1, W2, b2):
    src = edge_index[0]
    dst = edge_index[1]
    pad = EP - E
    padi = jnp.arange(pad, dtype=jnp.int32)
    src_g = jnp.concatenate([src, padi % N])
    src_d = jnp.concatenate([src, N + (padi % 112)])
    dst_d = jnp.concatenate([dst, N + (padi % 112)])

    z128 = jnp.zeros((RPT, D), jnp.float32)
    iota_h = jnp.arange(NROW, dtype=jnp.int32)
    b1_2d = b1.reshape(1, D)
    b2_2d = b2.reshape(1, D)

    src_g3 = src_g.reshape(NW, NCH, C)
    dst_d3 = dst_d.reshape(NW, NCH, C)

    deg_s, deg_d = _deg_kernel(src_d, dst_d, z128, iota_h)
    h1s, dsi, ddi = _scale(x, W1, deg_s.reshape(NC, NROW * 128, 1),
                           deg_d.reshape(NC, NROW * 128, 1))
    agg1 = _scat_kernel(h1s, src_g3, dst_d3, z128)
    h2s = _layer2(agg1, ddi, b1_2d, W2, dsi)
    agg2 = _scat_kernel(h2s, src_g3, dst_d3, z128)
    return _final(agg2, ddi, b2_2d)
